# baseline (speedup 1.0000x reference)
import jax
import jax.numpy as jnp
from jax.experimental import pallas as pl

BLOCK = 1024


def _bcast_kernel(table_ref, out_ref):
    out_ref[...] = jnp.broadcast_to(table_ref[...][None, :, :], out_ref.shape)


def kernel(inputs, pos_table):
    batch, n_seq = inputs.shape
    d_model = pos_table.shape[1]
    grid = (n_seq // BLOCK,)
    return pl.pallas_call(
        _bcast_kernel,
        grid=grid,
        in_specs=[pl.BlockSpec((BLOCK, d_model), lambda i: (i, 0))],
        out_specs=pl.BlockSpec((batch, BLOCK, d_model), lambda i: (0, i, 0)),
        out_shape=jax.ShapeDtypeStruct((batch, n_seq, d_model), pos_table.dtype),
    )(pos_table)
